# baseline (speedup 1.0000x reference)
import jax
import jax.numpy as jnp
from jax.experimental import pallas as pl
from jax.experimental.pallas import tpu as pltpu

N = 32768
H = 16
D = 64
S = 8
L = N // S
K = min(N, max(4096, 16))
RCH = 512


def _body(t_ref, nl_ref, q_ref, out_ref, rel_ref, sc_ref):
    p = pl.program_id(0)

    @pl.when(p < S)
    def _rel_phase():
        ones = jnp.ones((1, D), jnp.float32)
        parts = []
        for u in range(L // RCH):
            q_u = q_ref[0, u * RCH:(u + 1) * RCH, :]
            sq = q_u * q_u
            parts.append(jax.lax.dot_general(
                ones, sq, (((1,), (1,)), ((), ())),
                precision=jax.lax.Precision.HIGHEST,
                preferred_element_type=jnp.float32))
        mag2 = jnp.concatenate(parts, axis=1)
        t_p = t_ref[:, pl.ds(p * L, L)]
        nll_p = nl_ref[H - 1:H, pl.ds(p * L, L)]
        rel_ref[p] = jnp.sqrt(mag2) * jnp.exp(nll_p * t_p)

    @pl.when(p == S)
    def _search():
        rel = rel_ref[:, 0, :]
        relbits = jax.lax.bitcast_convert_type(rel, jnp.int32)

        def vsearch(i, lo):
            cand = lo | (jnp.int32(1) << (jnp.int32(30) - i))
            cnt = jnp.sum((relbits >= cand).astype(jnp.int32))
            return jnp.where(cnt >= K, cand, lo)

        taubits = jax.lax.fori_loop(0, 31, vsearch, jnp.int32(0))

        cnt_gt = jnp.sum((relbits > taubits).astype(jnp.int32))
        need = K - cnt_gt
        eq = relbits == taubits
        idx = (jax.lax.broadcasted_iota(jnp.int32, (S, L), 0) * L
               + jax.lax.broadcasted_iota(jnp.int32, (S, L), 1))

        def isearch(i, m):
            cand = m | (jnp.int32(1) << (jnp.int32(14) - i))
            cnt = jnp.sum(jnp.where(eq & (idx < cand), 1, 0))
            return jnp.where(cnt < need, cand, m)

        mlow = jax.lax.fori_loop(0, 15, isearch, jnp.int32(0))
        sc_ref[0] = taubits
        sc_ref[1] = mlow

    @pl.when(p >= S)
    def _mm_phase():
        c = p - S
        tau = sc_ref[0]
        mlow = sc_ref[1]
        bits = jax.lax.bitcast_convert_type(rel_ref[c], jnp.int32)
        idxc = c * L + jax.lax.broadcasted_iota(jnp.int32, (1, L), 1)
        mk = (bits > tau) | ((bits == tau) & (idxc <= mlow))
        t_c = t_ref[:, pl.ds(c * L, L)]
        nl_c = nl_ref[:, pl.ds(c * L, L)]
        w = jnp.where(mk, jnp.exp(nl_c * t_c), 0.0)
        part = jnp.dot(w, q_ref[0], preferred_element_type=jnp.float32)

        @pl.when(c == 0)
        def _init():
            out_ref[0] = part

        @pl.when(c > 0)
        def _acc():
            out_ref[0] = out_ref[0] + part


def kernel(time, negative_lambdas, quantities, top_k, min_tokens_to_keep):
    nl2 = negative_lambdas.reshape(H, N)
    q3 = quantities.reshape(S, L, D)
    return pl.pallas_call(
        _body,
        grid=(2 * S,),
        in_specs=[
            pl.BlockSpec((1, N), lambda p: (0, 0)),
            pl.BlockSpec((H, N), lambda p: (0, 0)),
            pl.BlockSpec((1, L, D), lambda p: (jnp.where(p < S, p, p - S), 0, 0)),
        ],
        out_specs=pl.BlockSpec((1, H, D), lambda p: (0, 0, 0)),
        out_shape=jax.ShapeDtypeStruct((1, H, D), jnp.float32),
        scratch_shapes=[pltpu.VMEM((S, 1, L), jnp.float32),
                        pltpu.SMEM((2,), jnp.int32)],
    )(time, nl2, q3)


# speedup vs baseline: 1.6659x; 1.6659x over previous
import jax
import jax.numpy as jnp
from jax.experimental import pallas as pl
from jax.experimental.pallas import tpu as pltpu

N = 32768
H = 16
D = 64
S = 8
L = N // S
K = min(N, max(4096, 16))


def _body(t_ref, nl_ref, qt_ref, out_ref, rel_ref, sc_ref):
    p = pl.program_id(0)

    @pl.when(p < S)
    def _rel_phase():
        qb = qt_ref[:, pl.ds(p * L, L)]
        sq = qb * qb
        mag2 = jax.lax.dot_general(
            jnp.ones((1, D), jnp.float32), sq, (((1,), (0,)), ((), ())),
            precision=jax.lax.Precision.HIGHEST,
            preferred_element_type=jnp.float32)
        t_p = t_ref[:, pl.ds(p * L, L)]
        nll_p = nl_ref[H - 1:H, pl.ds(p * L, L)]
        rel_ref[p] = jnp.sqrt(mag2) * jnp.exp(nll_p * t_p)

    @pl.when(p == S)
    def _search():
        rel = rel_ref[:, 0, :]
        relbits = jax.lax.bitcast_convert_type(rel, jnp.int32)

        def count_ge(cand):
            return jnp.sum((relbits >= cand).astype(jnp.int32))

        lo = jnp.where(count_ge(jnp.int32(1) << 30) >= K,
                       jnp.int32(1) << 30, jnp.int32(0))
        for d in range(9, -1, -1):
            step = jnp.int32(8 ** d)
            jpick = jnp.int32(0)
            for j in range(1, 8):
                jpick = jpick + (count_ge(lo + j * step) >= K).astype(jnp.int32)
            lo = lo + jpick * step
        taubits = lo

        need = K - count_ge(taubits + 1)
        eq = relbits == taubits
        idx = (jax.lax.broadcasted_iota(jnp.int32, (S, L), 0) * L
               + jax.lax.broadcasted_iota(jnp.int32, (S, L), 1))

        def count_tie_below(cand):
            return jnp.sum((eq & (idx < cand)).astype(jnp.int32))

        m = jnp.int32(0)
        for d in range(4, -1, -1):
            step = jnp.int32(8 ** d)
            jpick = jnp.int32(0)
            for j in range(1, 8):
                jpick = jpick + (count_tie_below(m + j * step) < need).astype(jnp.int32)
            m = m + jpick * step
        sc_ref[0] = taubits
        sc_ref[1] = m

    @pl.when(p >= S)
    def _mm_phase():
        c = p - S
        tau = sc_ref[0]
        mlow = sc_ref[1]
        bits = jax.lax.bitcast_convert_type(rel_ref[c], jnp.int32)
        idxc = c * L + jax.lax.broadcasted_iota(jnp.int32, (1, L), 1)
        mk = (bits > tau) | ((bits == tau) & (idxc <= mlow))
        t_c = t_ref[:, pl.ds(c * L, L)]
        nl_c = nl_ref[:, pl.ds(c * L, L)]
        w = jnp.where(mk, jnp.exp(nl_c * t_c), 0.0)
        qb = qt_ref[:, pl.ds(c * L, L)]
        part = jax.lax.dot_general(
            w, qb, (((1,), (1,)), ((), ())),
            preferred_element_type=jnp.float32)

        @pl.when(c == 0)
        def _init():
            out_ref[0] = part

        @pl.when(c > 0)
        def _acc():
            out_ref[0] = out_ref[0] + part


def kernel(time, negative_lambdas, quantities, top_k, min_tokens_to_keep):
    nl2 = negative_lambdas.reshape(H, N)
    qt = quantities.transpose(0, 2, 1).reshape(D, N)
    return pl.pallas_call(
        _body,
        grid=(2 * S,),
        in_specs=[
            pl.BlockSpec((1, N), lambda p: (0, 0)),
            pl.BlockSpec((H, N), lambda p: (0, 0)),
            pl.BlockSpec((D, N), lambda p: (0, 0)),
        ],
        out_specs=pl.BlockSpec((1, H, D), lambda p: (0, 0, 0)),
        out_shape=jax.ShapeDtypeStruct((1, H, D), jnp.float32),
        scratch_shapes=[pltpu.VMEM((S, 1, L), jnp.float32),
                        pltpu.SMEM((2,), jnp.int32)],
    )(time, nl2, qt)


# speedup vs baseline: 1.8069x; 1.0847x over previous
import jax
import jax.numpy as jnp
from jax.experimental import pallas as pl
from jax.experimental.pallas import tpu as pltpu

N = 32768
H = 16
D = 64
S = 8
L = N // S
K = min(N, max(4096, 16))


def _body(t_ref, nl_ref, qt_ref, out_ref, rel_ref, sc_ref):
    p = pl.program_id(0)

    @pl.when(p < S)
    def _rel_phase():
        SUB = 512
        for u in range(L // SUB):
            off = p * L + u * SUB
            acc = None
            for slab in range(D // 8):
                qs = qt_ref[pl.ds(slab * 8, 8), pl.ds(off, SUB)]
                sq = qs * qs
                acc = sq if acc is None else acc + sq
            mag2 = jnp.sum(acc, axis=0, keepdims=True)
            t_p = t_ref[:, pl.ds(off, SUB)]
            nll_p = nl_ref[H - 1:H, pl.ds(off, SUB)]
            rel_ref[p, :, pl.ds(u * SUB, SUB)] = (
                jnp.sqrt(mag2) * jnp.exp(nll_p * t_p))

    @pl.when(p == S)
    def _search():
        rel = rel_ref[:, 0, :]
        relbits = jax.lax.bitcast_convert_type(rel, jnp.int32)

        def count_ge(cand):
            return jnp.sum((relbits >= cand).astype(jnp.int32))

        lo = jnp.where(count_ge(jnp.int32(1) << 30) >= K,
                       jnp.int32(1) << 30, jnp.int32(0))
        for d in range(9, -1, -1):
            step = jnp.int32(8 ** d)
            jpick = jnp.int32(0)
            for j in range(1, 8):
                jpick = jpick + (count_ge(lo + j * step) >= K).astype(jnp.int32)
            lo = lo + jpick * step
        taubits = lo

        need = K - count_ge(taubits + 1)
        eq = relbits == taubits
        idx = (jax.lax.broadcasted_iota(jnp.int32, (S, L), 0) * L
               + jax.lax.broadcasted_iota(jnp.int32, (S, L), 1))

        def count_tie_below(cand):
            return jnp.sum((eq & (idx < cand)).astype(jnp.int32))

        m = jnp.int32(0)
        for d in range(4, -1, -1):
            step = jnp.int32(8 ** d)
            jpick = jnp.int32(0)
            for j in range(1, 8):
                jpick = jpick + (count_tie_below(m + j * step) < need).astype(jnp.int32)
            m = m + jpick * step
        sc_ref[0] = taubits
        sc_ref[1] = m

    @pl.when(p >= S)
    def _mm_phase():
        c = p - S
        tau = sc_ref[0]
        mlow = sc_ref[1]
        SUB = 512
        part = jnp.zeros((H, D), jnp.float32)
        for u in range(L // SUB):
            off = c * L + u * SUB
            bits = jax.lax.bitcast_convert_type(
                rel_ref[c, :, pl.ds(u * SUB, SUB)], jnp.int32)
            idxc = off + jax.lax.broadcasted_iota(jnp.int32, (1, SUB), 1)
            mk = (bits > tau) | ((bits == tau) & (idxc <= mlow))
            t_c = t_ref[:, pl.ds(off, SUB)]
            nl_c = nl_ref[:, pl.ds(off, SUB)]
            w = jnp.where(mk, jnp.exp(nl_c * t_c), 0.0)
            qb = qt_ref[:, pl.ds(off, SUB)]
            part = part + jax.lax.dot_general(
                w, qb, (((1,), (1,)), ((), ())),
                preferred_element_type=jnp.float32)

        @pl.when(c == 0)
        def _init():
            out_ref[0] = part

        @pl.when(c > 0)
        def _acc():
            out_ref[0] = out_ref[0] + part


def kernel(time, negative_lambdas, quantities, top_k, min_tokens_to_keep):
    nl2 = negative_lambdas.reshape(H, N)
    qt = quantities.transpose(0, 2, 1).reshape(D, N)
    return pl.pallas_call(
        _body,
        grid=(2 * S,),
        in_specs=[
            pl.BlockSpec((1, N), lambda p: (0, 0)),
            pl.BlockSpec((H, N), lambda p: (0, 0)),
            pl.BlockSpec((D, N), lambda p: (0, 0)),
        ],
        out_specs=pl.BlockSpec((1, H, D), lambda p: (0, 0, 0)),
        out_shape=jax.ShapeDtypeStruct((1, H, D), jnp.float32),
        scratch_shapes=[pltpu.VMEM((S, 1, L), jnp.float32),
                        pltpu.SMEM((2,), jnp.int32)],
    )(time, nl2, qt)
